# Initial kernel scaffold; baseline (speedup 1.0000x reference)
import jax
import jax.numpy as jnp
from jax.experimental import pallas as pl


def kernel(x, edge_index, graph_ids, target, dist_dicts, W1, al1, ar1, b1, W2, al2, ar2, b2, Wa, ba, Wp, bp, Wd, bd, Wf1, bf1, Wf2, bf2, Wo, bo):
    raise NotImplementedError("write your pallas kernel here")


# speedup vs baseline: 17.6941x; 17.6941x over previous
import functools

import jax
import jax.numpy as jnp
from jax import lax
from jax.experimental import pallas as pl
from jax.experimental.pallas import tpu as pltpu
from jax.experimental.pallas import tpu_sc as plsc

N = 50000
E = 800000
NB = 1024
H1, F1, FP1 = 10, 74, 80
D1 = H1 * FP1
F2, FP2 = 128, 144

NC, NS = 2, 16
NBLK = 400

R1, ACC1, RG1 = 1024, 1040, 25
R2, ACC2, RG2 = 6400, 6416, 4
OUT1_ROWS = 2 * RG1 * R1
OUT2_ROWS = 2 * RG2 * R2
EP = 819200
ESPAN = EP // NS
CHUNK = 2048
NCHUNK = ESPAN // CHUNK
BT = 32
CB = CHUNK + BT
C3 = 128

_mesh = plsc.VectorSubcoreMesh(
    core_axis_name="c", subcore_axis_name="s", num_cores=NC, num_subcores=NS)


def _tc1_body(x_ref, w1_ref, al_ref, ar_ref, hpad_ref, el_ref, er_ref):
    hb = jnp.dot(x_ref[...], w1_ref[...], preferred_element_type=jnp.float32)
    el_ref[...] = jnp.dot(hb, al_ref[...], preferred_element_type=jnp.float32)
    er_ref[...] = jnp.dot(hb, ar_ref[...], preferred_element_type=jnp.float32)
    rows = hb.shape[0]
    ones = jnp.ones((rows, 1), jnp.float32)
    zer = jnp.zeros((rows, FP1 - F1 - 1), jnp.float32)
    parts = []
    for h in range(H1):
        parts += [hb[:, h * F1:(h + 1) * F1], ones, zer]
    hpad_ref[...] = jnp.concatenate(parts, axis=1)


def _tc2_body(o1_ref, b1_ref, w2_ref, al2_ref, ar2_ref,
              hp2_ref, el2_ref, er2_ref):
    blk = o1_ref[...]
    parts = []
    for h in range(H1):
        f = blk[:, h * FP1:h * FP1 + F1]
        d = blk[:, h * FP1 + F1:h * FP1 + F1 + 1]
        parts.append(f / (d + 1e-9))
    h1 = jnp.concatenate(parts, axis=1) + b1_ref[...]
    h1 = jnp.where(h1 > 0, h1, jnp.exp(jnp.minimum(h1, 0.0)) - 1.0)
    h2p = jnp.dot(h1, w2_ref[...], preferred_element_type=jnp.float32)
    el2 = jnp.sum(h2p * al2_ref[...], axis=1, keepdims=True)
    er2 = jnp.sum(h2p * ar2_ref[...], axis=1, keepdims=True)
    rows = blk.shape[0]
    z15 = jnp.zeros((rows, 15), jnp.float32)
    el2_ref[...] = jnp.concatenate([el2, z15], axis=1)
    er2_ref[...] = jnp.concatenate([er2, z15], axis=1)
    hp2_ref[...] = jnp.concatenate(
        [h2p, jnp.ones((rows, 1), jnp.float32), z15], axis=1)


def _tc4_body(t_ref, d_ref, xm_ref, wp_ref, bp_ref, wd_ref, bd_ref,
              wf1_ref, bf1_ref, wf2_ref, bf2_ref, wo_ref, bo_ref, o_ref):
    xp = jnp.dot(t_ref[...], wp_ref[...],
                 preferred_element_type=jnp.float32) + bp_ref[...]
    xd = jnp.dot(d_ref[...], wd_ref[...],
                 preferred_element_type=jnp.float32) + bd_ref[...]
    xc = jnp.concatenate([xp, xm_ref[...], xd], axis=1)
    h = jnp.maximum(jnp.dot(xc, wf1_ref[...],
                            preferred_element_type=jnp.float32)
                    + bf1_ref[...], 0.0)
    h = jnp.maximum(jnp.dot(h, wf2_ref[...],
                            preferred_element_type=jnp.float32)
                    + bf2_ref[...], 0.0)
    o_ref[...] = jnp.dot(h, wo_ref[...],
                         preferred_element_type=jnp.float32) + bo_ref[...]


def _tc1(x, W1, AL1, AR1):
    grid = (N // NBLK,)
    return pl.pallas_call(
        _tc1_body,
        grid=grid,
        in_specs=[
            pl.BlockSpec((NBLK, F1 * 1), lambda i: (i, 0)),
            pl.BlockSpec((F1, H1 * F1), lambda i: (0, 0)),
            pl.BlockSpec((H1 * F1, 16), lambda i: (0, 0)),
            pl.BlockSpec((H1 * F1, 16), lambda i: (0, 0)),
        ],
        out_specs=[
            pl.BlockSpec((NBLK, D1), lambda i: (i, 0)),
            pl.BlockSpec((NBLK, 16), lambda i: (i, 0)),
            pl.BlockSpec((NBLK, 16), lambda i: (i, 0)),
        ],
        out_shape=[
            jax.ShapeDtypeStruct((N, D1), jnp.float32),
            jax.ShapeDtypeStruct((N, 16), jnp.float32),
            jax.ShapeDtypeStruct((N, 16), jnp.float32),
        ],
    )(x, W1, AL1, AR1)


def _tc2(o1, b1r, W2, al2r, ar2r):
    grid = (N // NBLK,)
    return pl.pallas_call(
        _tc2_body,
        grid=grid,
        in_specs=[
            pl.BlockSpec((NBLK, D1), lambda i: (i, 0)),
            pl.BlockSpec((1, H1 * F1), lambda i: (0, 0)),
            pl.BlockSpec((H1 * F1, F2), lambda i: (0, 0)),
            pl.BlockSpec((1, F2), lambda i: (0, 0)),
            pl.BlockSpec((1, F2), lambda i: (0, 0)),
        ],
        out_specs=[
            pl.BlockSpec((NBLK, FP2), lambda i: (i, 0)),
            pl.BlockSpec((NBLK, 16), lambda i: (i, 0)),
            pl.BlockSpec((NBLK, 16), lambda i: (i, 0)),
        ],
        out_shape=[
            jax.ShapeDtypeStruct((N, FP2), jnp.float32),
            jax.ShapeDtypeStruct((N, 16), jnp.float32),
            jax.ShapeDtypeStruct((N, 16), jnp.float32),
        ],
    )(o1, b1r, W2, al2r, ar2r)


def _tc4(target, dist, xmol, Wp, bpr, Wd, bdr, Wf1, bf1r, Wf2, bf2r, Wop, bop):
    grid = (NB // 128,)
    return pl.pallas_call(
        _tc4_body,
        grid=grid,
        in_specs=[
            pl.BlockSpec((128, 1024), lambda i: (i, 0)),
            pl.BlockSpec((128, 4096), lambda i: (i, 0)),
            pl.BlockSpec((128, 256), lambda i: (i, 0)),
            pl.BlockSpec((1024, 256), lambda i: (0, 0)),
            pl.BlockSpec((1, 256), lambda i: (0, 0)),
            pl.BlockSpec((4096, 256), lambda i: (0, 0)),
            pl.BlockSpec((1, 256), lambda i: (0, 0)),
            pl.BlockSpec((768, 1024), lambda i: (0, 0)),
            pl.BlockSpec((1, 1024), lambda i: (0, 0)),
            pl.BlockSpec((1024, 256), lambda i: (0, 0)),
            pl.BlockSpec((1, 256), lambda i: (0, 0)),
            pl.BlockSpec((256, 128), lambda i: (0, 0)),
            pl.BlockSpec((1, 128), lambda i: (0, 0)),
        ],
        out_specs=pl.BlockSpec((128, 128), lambda i: (i, 0)),
        out_shape=jax.ShapeDtypeStruct((NB, 128), jnp.float32),
    )(target, dist, xmol, Wp, bpr, Wd, bdr, Wf1, bf1r, Wf2, bf2r, Wop, bop)


def _make_gat_sc(fp, nheads, r_rows, acc_rows, nranges, out_rows):
    nvr = fp // 16
    hw = fp // nheads
    zrows = r_rows // 16
    crows = r_rows // 16

    @functools.partial(
        pl.kernel, mesh=_mesh,
        compiler_params=pltpu.CompilerParams(needs_layout_passes=False, use_tc_tiling_on_sc=False),
        out_type=jax.ShapeDtypeStruct((out_rows, fp), jnp.float32),
        scratch_types=[
            pltpu.VMEM((CHUNK,), jnp.int32),
            pltpu.VMEM((CHUNK,), jnp.int32),
            pltpu.VMEM((CB,), jnp.int32),
            pltpu.VMEM((CB,), jnp.int32),
            pltpu.VMEM((2, BT), jnp.int32),
            pltpu.VMEM((2, BT), jnp.int32),
            pltpu.VMEM((2, BT), jnp.int32),
            pltpu.VMEM((2, BT, 16), jnp.float32),
            pltpu.VMEM((2, BT, 16), jnp.float32),
            pltpu.VMEM((2, BT, 16), jnp.float32),
            pltpu.VMEM((2, BT, fp), jnp.float32),
            pltpu.VMEM((8, fp), jnp.float32),
            pltpu.VMEM_SHARED((acc_rows, fp), jnp.float32),
            pltpu.SemaphoreType.DMA((2,)),
            pltpu.SemaphoreType.DMA((2,)),
            pltpu.SemaphoreType.DMA((2,)),
        ],
    )
    def gat(src_hbm, dst_hbm, el_hbm, er_hbm, feat_hbm, out_hbm,
            csv, cdv, csrc, cloc, bs, bd, bl,
            elr, err, ee, rows, zbuf, acc, sem_e, sem_r, sem_f):
        cid = lax.axis_index("c")
        sid = lax.axis_index("s")
        ebase = sid * ESPAN

        def zb(i, _):
            for v in range(nvr):
                zbuf[i, pl.ds(v * 16, 16)] = jnp.zeros((16,), jnp.float32)
            return 0
        lax.fori_loop(0, 8, zb, 0)

        def range_body(rg, _):
            lo = cid * (nranges * r_rows) + rg * r_rows

            def zcopy(z, _):
                pltpu.sync_copy(zbuf, acc.at[pl.ds(sid * zrows + z * 8, 8)])
                return 0
            lax.fori_loop(0, zrows // 8, zcopy, 0)
            plsc.subcore_barrier()

            def stage_a(i, b0):
                p = i & 1
                for k in range(BT // 16):
                    ksl = pl.ds(k * 16, 16)
                    t = cloc[pl.ds(b0 + k * 16, 16)]
                    bs[p, ksl] = csrc[pl.ds(b0 + k * 16, 16)]
                    bd[p, ksl] = t + lo
                    bl[p, ksl] = t
                pltpu.async_copy(el_hbm.at[bs.at[p]], elr.at[p], sem_e.at[p])
                pltpu.async_copy(er_hbm.at[bd.at[p]], err.at[p], sem_r.at[p])
                pltpu.async_copy(feat_hbm.at[bs.at[p]], rows.at[p],
                                 sem_f.at[p])

            def stage_b(i):
                p = i & 1
                pltpu.make_async_copy(
                    el_hbm.at[pl.ds(0, BT)], elr.at[p], sem_e.at[p]).wait()
                pltpu.make_async_copy(
                    er_hbm.at[pl.ds(0, BT)], err.at[p], sem_r.at[p]).wait()

                def eeb(r, _):
                    s = elr[p, r, :] + err[p, r, :]
                    s = jnp.where(s > 0, s, 0.2 * s)
                    ee[p, r, :] = jnp.exp(s)
                    return 0
                lax.fori_loop(0, BT, eeb, 0)
                pltpu.make_async_copy(
                    feat_hbm.at[pl.ds(0, BT)], rows.at[p], sem_f.at[p]).wait()

                def scale(r, _):
                    for h in range(nheads):
                        coef = plsc.load_gather(
                            ee, [jnp.full((16,), p, jnp.int32),
                                 jnp.full((16,), r, jnp.int32),
                                 jnp.full((16,), h, jnp.int32)])
                        for v in range(hw // 16):
                            fsl = pl.ds(h * hw + v * 16, 16)
                            rows[p, r, fsl] = rows[p, r, fsl] * coef
                    return 0
                lax.fori_loop(0, BT, scale, 0)
                pltpu.sync_copy(rows.at[p], acc.at[bl.at[p]], add=True)

            def chunk_body(c, carry):
                pos, gb = carry
                coff = ebase + c * CHUNK
                pltpu.sync_copy(src_hbm.at[pl.ds(coff, CHUNK)], csv)
                pltpu.sync_copy(dst_hbm.at[pl.ds(coff, CHUNK)], cdv)

                def scan(i, p_):
                    sl = pl.ds(i * 16, 16)
                    sv = csv[sl]
                    dv = cdv[sl]
                    m = (dv >= lo) & (dv < lo + r_rows)
                    mi = jnp.where(m, jnp.int32(1), jnp.int32(0))
                    csum = plsc.cumsum(mi)
                    idx = p_ + csum - mi
                    plsc.store_scatter(csrc, [idx], sv, mask=m)
                    plsc.store_scatter(cloc, [idx], dv - lo, mask=m)
                    return p_ + csum[15]
                pos = lax.fori_loop(0, CHUNK // 16, scan, pos)
                nbf = pos // BT

                def batch(j, g):
                    stage_a(g, j * BT)

                    @pl.when(g > 0)
                    def _():
                        stage_b(g - 1)
                    return g + 1
                gb = lax.fori_loop(0, nbf, batch, gb)

                @pl.when(nbf > 0)
                def _():
                    for k in range(BT // 16):
                        ksl = pl.ds(k * 16, 16)
                        bsl = pl.ds(nbf * BT + k * 16, 16)
                        csrc[ksl] = csrc[bsl]
                        cloc[ksl] = cloc[bsl]
                return (pos - nbf * BT, gb)
            pos, gb = lax.fori_loop(0, NCHUNK, chunk_body,
                                    (jnp.int32(0), jnp.int32(0)))

            @pl.when(pos > 0)
            def _():
                zi = jnp.zeros((16,), jnp.int32)
                dumpv = jnp.full((16,), r_rows, jnp.int32)
                for k in range(BT // 16):
                    psl = pl.ds(pos + k * 16, 16)
                    csrc[psl] = zi
                    cloc[psl] = dumpv
                stage_a(gb, 0)

            @pl.when(gb > 0)
            def _():
                stage_b(gb - 1)

            @pl.when(pos > 0)
            def _():
                stage_b(gb)
            plsc.subcore_barrier()

            obase = pl.multiple_of(
                (cid * nranges + rg) * r_rows + sid * crows, 8)
            pltpu.sync_copy(acc.at[pl.ds(sid * crows, crows)],
                            out_hbm.at[pl.ds(obase, crows)])
            plsc.subcore_barrier()
            return 0
        lax.fori_loop(0, nranges, range_body, 0)
    return gat


_gat1 = _make_gat_sc(D1, H1, R1, ACC1, RG1, OUT1_ROWS)
_gat2 = _make_gat_sc(FP2, 1, R2, ACC2, RG2, OUT2_ROWS)


@functools.partial(
    pl.kernel, mesh=_mesh,
    compiler_params=pltpu.CompilerParams(needs_layout_passes=False, use_tc_tiling_on_sc=False),
    out_type=jax.ShapeDtypeStruct((NB, 2 * F2), jnp.float32),
    scratch_types=[
        pltpu.VMEM((C3, FP2), jnp.float32),
        pltpu.VMEM((32, 2 * F2), jnp.float32),
        pltpu.VMEM((F2,), jnp.float32),
        pltpu.VMEM((F2,), jnp.float32),
        pltpu.VMEM((16,), jnp.float32),
        pltpu.VMEM((48,), jnp.int32),
    ],
)
def _readout(out2_hbm, off_hbm, b2_hbm, wa_hbm, ba_hbm, xmol_hbm,
             nodebuf, xout, b2v, wav, bav, offs):
    cid = lax.axis_index("c")
    sid = lax.axis_index("s")
    w = sid * NC + cid
    pltpu.sync_copy(b2_hbm, b2v)
    pltpu.sync_copy(wa_hbm, wav)
    pltpu.sync_copy(ba_hbm, bav)
    pltpu.sync_copy(off_hbm.at[pl.ds(w * 32, 48)], offs)
    bvec = bav[...]

    def graph_body(g, _):
        start = plsc.load_gather(
            offs, [jnp.full((16,), g, jnp.int32)])[0]
        end = plsc.load_gather(
            offs, [jnp.full((16,), g + 1, jnp.int32)])[0]
        nn = end - start
        a8 = (start // 8) * 8
        nch = (end - a8 + C3 - 1) // C3

        def chunk(k, carry):
            gs, gm = carry
            cs = pl.multiple_of(a8 + k * C3, 8)
            rlo = jnp.maximum(start - cs, 0)
            rhi = jnp.minimum(end - cs, C3)
            pltpu.sync_copy(out2_hbm.at[pl.ds(cs, C3)], nodebuf)

            def node(r, cr):
                ngs, ngm = cr
                den = plsc.load_gather(
                    nodebuf, [jnp.full((16,), r, jnp.int32),
                              jnp.full((16,), F2, jnp.int32)])
                hv = []
                accw = jnp.zeros((16,), jnp.float32)
                for v in range(F2 // 16):
                    xv = (nodebuf[r, pl.ds(v * 16, 16)] / (den + 1e-9)
                          + b2v[pl.ds(v * 16, 16)])
                    hv.append(xv)
                    accw = accw + xv * wav[pl.ds(v * 16, 16)]
                ssum = jnp.sum(accw)
                sv = jnp.full((16,), ssum, jnp.float32)
                wn = 1.0 / (1.0 + jnp.exp(-(sv + bvec)))
                ngs = tuple(ngs[v] + hv[v] * wn for v in range(F2 // 16))
                ngm = tuple(jnp.maximum(ngm[v], hv[v])
                            for v in range(F2 // 16))
                return (ngs, ngm)
            return lax.fori_loop(rlo, rhi, node, (gs, gm))

        init = (tuple(jnp.zeros((16,), jnp.float32) for _ in range(F2 // 16)),
                tuple(jnp.full((16,), -3.0e38, jnp.float32)
                      for _ in range(F2 // 16)))
        gsum, gmax = lax.fori_loop(0, nch, chunk, init)
        nnv = jnp.full((16,), nn, jnp.int32)
        for v in range(F2 // 16):
            xout[g, pl.ds(v * 16, 16)] = gsum[v]
            gmv = jnp.where(nnv > 0, gmax[v], jnp.zeros((16,), jnp.float32))
            xout[g, pl.ds(F2 + v * 16, 16)] = gmv
        return 0
    lax.fori_loop(0, 32, graph_body, 0)
    pltpu.sync_copy(xout, xmol_hbm.at[pl.ds(w * 32, 32)])


def kernel(x, edge_index, graph_ids, target, dist_dicts,
           W1, al1, ar1, b1, W2, al2, ar2, b2, Wa, ba,
           Wp, bp, Wd, bd, Wf1, bf1, Wf2, bf2, Wo, bo):
    f32 = jnp.float32
    src = edge_index[0]
    dst = edge_index[1]
    srcp = jnp.concatenate([src, jnp.zeros((EP - E,), jnp.int32)])
    dstp = jnp.concatenate([dst, jnp.full((EP - E,), 60000, jnp.int32)])

    AL1 = jnp.zeros((H1 * F1, 16), f32)
    AR1 = jnp.zeros((H1 * F1, 16), f32)
    for h in range(H1):
        AL1 = AL1.at[h * F1:(h + 1) * F1, h].set(al1[h])
        AR1 = AR1.at[h * F1:(h + 1) * F1, h].set(ar1[h])

    hpad1, el1, er1 = _tc1(x, W1, AL1, AR1)
    out1 = _gat1(srcp, dstp, el1, er1, hpad1)

    hpad2, el2, er2 = _tc2(out1[:N], b1.reshape(1, -1), W2,
                           al2.reshape(1, -1), ar2.reshape(1, -1))
    out2 = _gat2(srcp, dstp, el2, er2, hpad2)

    out2p = jnp.concatenate([out2[:N], jnp.zeros((C3, FP2), f32)], axis=0)
    offsets = jnp.searchsorted(
        graph_ids, jnp.arange(NB + 1, dtype=jnp.int32)).astype(jnp.int32)
    offp = jnp.concatenate([offsets, jnp.zeros((1040 - (NB + 1),), jnp.int32)])
    ba16 = jnp.full((16,), ba[0], f32)
    xmol = _readout(out2p, offp, b2, Wa.reshape(-1), ba16)

    Wop = jnp.zeros((256, 128), f32).at[:, 0].set(Wo[:, 0])
    bop = jnp.zeros((1, 128), f32).at[0, 0].set(bo[0])
    out = _tc4(target, dist_dicts, xmol, Wp, bp.reshape(1, -1), Wd,
               bd.reshape(1, -1), Wf1, bf1.reshape(1, -1), Wf2,
               bf2.reshape(1, -1), Wop, bop)
    return out[:, :1]
